# baseline (speedup 1.0000x reference)
import jax
import jax.numpy as jnp
from jax import lax
from jax.experimental import pallas as pl
from jax.experimental.pallas import tpu as pltpu
from jax.experimental.pallas import tpu_sc as plsc

NC, NS, L = 2, 16, 16
NW = NC * NS
CH = 32


def _build(B, S, D):
    N = B * S
    rows_per_w = N // NW
    s_per_w = S // NW
    halves = s_per_w // CH
    n_chunks = B * halves
    groups = D // L

    mesh = plsc.VectorSubcoreMesh(core_axis_name="c", subcore_axis_name="s")

    def body(tok_hbm, pos_hbm, seg_hbm, xf_hbm, sfb_hbm, out_hbm,
             idx_v, st_b, d_b, pos_r0, pos_r1,
             tok_b0, tok_b1, lab_b0, lab_b1,
             gsem0, gsem1, lsem0, lsem1, ssem0, ssem1):
        wid = lax.axis_index("s") * NC + lax.axis_index("c")
        spos = wid * s_per_w
        tok_bufs = (tok_b0, tok_b1)
        lab_bufs = (lab_b0, lab_b1)
        gsems = (gsem0, gsem1)
        lsems = (lsem0, lsem1)
        ssems = (ssem0, ssem1)

        pos_res = (pos_r0, pos_r1)
        for h in range(halves):
            pltpu.sync_copy(pos_hbm.at[pl.ds(spos + h * CH, CH)], pos_res[h])
        for b in range(B):
            pltpu.sync_copy(xf_hbm.at[pl.ds(b * S + spos, s_per_w)],
                            idx_v.at[pl.ds(b * s_per_w, s_per_w)])
        pltpu.sync_copy(seg_hbm, st_b)
        for g in range(groups):
            sl = pl.ds(g * L, L)
            d_b[sl] = st_b[1, sl] - st_b[0, sl]

        def hbm_off(c):
            b, h = divmod(c, halves)
            return b * S + spos + h * CH

        def issue(c):
            s = c % 2
            dt = pltpu.async_copy(
                tok_hbm.at[idx_v.at[pl.ds((c // halves) * s_per_w
                                          + (c % halves) * CH, CH)]],
                tok_bufs[s], gsems[s])
            dl = pltpu.async_copy(
                sfb_hbm.at[pl.ds(hbm_off(c), CH)], lab_bufs[s], lsems[s])
            return dt, dl

        in_flight = {0: issue(0)}
        stores = {}
        for c in range(n_chunks):
            s = c % 2
            h = c % halves
            if c + 1 < n_chunks:
                if c >= 1:
                    stores.pop(c - 1).wait()
                in_flight[c + 1] = issue(c + 1)
            for d in in_flight.pop(c):
                d.wait()
            tb, lb, prb = tok_bufs[s], lab_bufs[s], pos_res[h]

            half = groups // 2
            for gh in range(2):
                consts = []
                for g in range(gh * half, (gh + 1) * half):
                    slg = pl.ds(g * L, L)
                    consts.append((slg, st_b[0, slg], d_b[slg]))

                def row_add(r):
                    lf = lb[r, :]
                    for slg, st0g, d01g in consts:
                        tb[r, slg] = (tb[r, slg] + prb[r, slg]
                                      + (st0g + lf * d01g))

                plsc.parallel_loop(0, CH, 1, unroll=2)(row_add)

            stores[c] = pltpu.async_copy(
                tb, out_hbm.at[pl.ds(hbm_off(c), CH)], ssems[s])
        stores.pop(n_chunks - 2).wait()
        stores.pop(n_chunks - 1).wait()

    return pl.kernel(
        body,
        out_type=jax.ShapeDtypeStruct((N, D), jnp.float32),
        mesh=mesh,
        scratch_types=[
            pltpu.VMEM((rows_per_w,), jnp.int32),
            pltpu.VMEM((2, D), jnp.float32),
            pltpu.VMEM((D,), jnp.float32),
            pltpu.VMEM((CH, D), jnp.float32),
            pltpu.VMEM((CH, D), jnp.float32),
            pltpu.VMEM((CH, D), jnp.float32),
            pltpu.VMEM((CH, D), jnp.float32),
            pltpu.VMEM((CH, L), jnp.float32),
            pltpu.VMEM((CH, L), jnp.float32),
            pltpu.SemaphoreType.DMA,
            pltpu.SemaphoreType.DMA,
            pltpu.SemaphoreType.DMA,
            pltpu.SemaphoreType.DMA,
            pltpu.SemaphoreType.DMA,
            pltpu.SemaphoreType.DMA,
        ],
    )


def kernel(x, segment_label, tok_table, seg_table, pos_encoding):
    B, S = x.shape
    D = tok_table.shape[1]
    N = B * S
    xf = x.reshape(N).astype(jnp.int32)
    sfb = jnp.broadcast_to(
        segment_label.reshape(N, 1).astype(jnp.float32), (N, L))
    out = _build(B, S, D)(tok_table, pos_encoding, seg_table, xf, sfb)
    return out.reshape(B, S, D)


# speedup vs baseline: 1.1684x; 1.1684x over previous
import jax
import jax.numpy as jnp
from jax import lax
from jax.experimental import pallas as pl
from jax.experimental.pallas import tpu as pltpu
from jax.experimental.pallas import tpu_sc as plsc

NC, NS, L = 2, 16, 16
NW = NC * NS
CH = 32


def _build(N, S, D):
    rows_per_w = N // NW
    n_chunks = rows_per_w // CH
    groups = D // L

    mesh = plsc.VectorSubcoreMesh(core_axis_name="c", subcore_axis_name="s")

    def body(tok_hbm, pos_hbm, seg_hbm, xf_hbm, sfb_hbm, out_hbm,
             idx_v, st_b, d_b,
             tok_b0, tok_b1, pos_b0, pos_b1, lab_b0, lab_b1,
             gsem0, gsem1, psem0, psem1, lsem0, lsem1, ssem0, ssem1):
        wid = lax.axis_index("s") * NC + lax.axis_index("c")
        base = wid * rows_per_w
        spos = base % S
        tok_bufs = (tok_b0, tok_b1)
        pos_bufs = (pos_b0, pos_b1)
        lab_bufs = (lab_b0, lab_b1)
        gsems = (gsem0, gsem1)
        psems = (psem0, psem1)
        lsems = (lsem0, lsem1)
        ssems = (ssem0, ssem1)

        pltpu.sync_copy(xf_hbm.at[pl.ds(base, rows_per_w)], idx_v)

        def issue(c):
            s = c % 2
            dt = pltpu.async_copy(
                tok_hbm.at[idx_v.at[pl.ds(c * CH, CH)]], tok_bufs[s], gsems[s])
            dp = pltpu.async_copy(
                pos_hbm.at[pl.ds(spos + c * CH, CH)], pos_bufs[s], psems[s])
            dl = pltpu.async_copy(
                sfb_hbm.at[pl.ds(base + c * CH, CH)], lab_bufs[s], lsems[s])
            return dt, dp, dl

        in_flight = {0: issue(0)}

        pltpu.sync_copy(seg_hbm, st_b)
        for g in range(groups):
            sl = pl.ds(g * L, L)
            d_b[sl] = st_b[1, sl] - st_b[0, sl]

        stores = {}
        for c in range(n_chunks):
            s = c % 2
            if c + 1 < n_chunks:
                if c >= 1:
                    for d in stores.pop(c - 1):
                        d.wait()
                in_flight[c + 1] = issue(c + 1)
            for d in in_flight.pop(c):
                d.wait()
            tb, pb, lb = tok_bufs[s], pos_bufs[s], lab_bufs[s]

            half = groups // 2
            hrows = CH // 2
            chunk_stores = []
            for gh in range(2):
                consts = []
                for g in range(gh * half, (gh + 1) * half):
                    slg = pl.ds(g * L, L)
                    consts.append((slg, st_b[0, slg], d_b[slg]))

                def row_add(r):
                    lf = lb[r, :]
                    for slg, st0g, d01g in consts:
                        tb[r, slg] = (tb[r, slg] + pb[r, slg]
                                      + (st0g + lf * d01g))

                plsc.parallel_loop(gh * hrows, (gh + 1) * hrows, 1,
                                   unroll=2)(row_add)
                chunk_stores.append(pltpu.async_copy(
                    tb.at[pl.ds(gh * hrows, hrows)],
                    out_hbm.at[pl.ds(base + c * CH + gh * hrows, hrows)],
                    ssems[s]))
            stores[c] = chunk_stores
        for c in sorted(stores):
            for d in stores.pop(c):
                d.wait()

    return pl.kernel(
        body,
        out_type=jax.ShapeDtypeStruct((N, D), jnp.float32),
        mesh=mesh,
        scratch_types=[
            pltpu.VMEM((rows_per_w,), jnp.int32),
            pltpu.VMEM((2, D), jnp.float32),
            pltpu.VMEM((D,), jnp.float32),
            pltpu.VMEM((CH, D), jnp.float32),
            pltpu.VMEM((CH, D), jnp.float32),
            pltpu.VMEM((CH, D), jnp.float32),
            pltpu.VMEM((CH, D), jnp.float32),
            pltpu.VMEM((CH, L), jnp.float32),
            pltpu.VMEM((CH, L), jnp.float32),
            pltpu.SemaphoreType.DMA,
            pltpu.SemaphoreType.DMA,
            pltpu.SemaphoreType.DMA,
            pltpu.SemaphoreType.DMA,
            pltpu.SemaphoreType.DMA,
            pltpu.SemaphoreType.DMA,
            pltpu.SemaphoreType.DMA,
            pltpu.SemaphoreType.DMA,
        ],
    )


def kernel(x, segment_label, tok_table, seg_table, pos_encoding):
    B, S = x.shape
    D = tok_table.shape[1]
    N = B * S
    xf = x.reshape(N).astype(jnp.int32)
    sfb = jnp.broadcast_to(
        segment_label.reshape(N, 1).astype(jnp.float32), (N, L))
    out = _build(N, S, D)(tok_table, pos_encoding, seg_table, xf, sfb)
    return out.reshape(B, S, D)
